# baseline (speedup 1.0000x reference)
import functools

import jax
import jax.numpy as jnp
from jax import lax
from jax.experimental import pallas as pl
from jax.experimental.pallas import tpu as pltpu
from jax.experimental.pallas import tpu_sc as plsc

B, N, C, K = 2, 4096, 64, 16
CP = 128
TILE = 1024
NEG_BIG = -1e30

_info = plsc.get_sparse_core_info()
NC, NS, L = _info.num_cores, _info.num_subcores, _info.num_lanes
NW = NC * NS
PTS_PER_W = (B * N) // NW
CHUNK = 16
GATHER = 128
NBUF = 2


def _topk_idx_body(x_rows_ref, x_all_ref, idx_ref):
        b = pl.program_id(0)
        xr = x_rows_ref[0]
        xa = x_all_ref[0]
        inner = lax.dot_general(xr, xa, (((1,), (1,)), ((), ())),
                                preferred_element_type=jnp.float32)
        xx_r = jnp.sum(xr * xr, axis=1, keepdims=True)
        xx_a = jnp.sum(xa * xa, axis=1).reshape(1, N)
        neg = 2.0 * inner - xx_r - xx_a
        colf = lax.broadcasted_iota(jnp.int32, (TILE, N), 1).astype(jnp.float32)
        BIGF = 1e9
        m = jnp.max(neg, axis=1, keepdims=True)
        picks = []
        for t in range(K):
            hit = neg == m
            amin_f = jnp.min(jnp.where(hit, colf, BIGF), axis=1,
                             keepdims=True)
            picks.append(amin_f)
            if t < K - 1:
                neg = jnp.where(hit, NEG_BIG, neg)
                m = jnp.max(neg, axis=1, keepdims=True)
        idx_f = jnp.minimum(jnp.concatenate(picks, axis=1), float(N - 1))
        idx_ref[0] = idx_f.astype(jnp.int32) + b * N


def _head_body(feat0_ref, feat1_ref, w_ref, gamma_ref, beta_ref, out_ref):
    feat = jnp.concatenate([feat0_ref[...][:, :C], feat1_ref[...][:, :C]],
                           axis=0)
    w = w_ref[...]
    y = lax.dot_general(feat, w, (((1,), (1,)), ((), ())),
                        preferred_element_type=jnp.float32)
    mean = jnp.mean(y, axis=0, keepdims=True)
    var = jnp.mean(y * y, axis=0, keepdims=True) - mean * mean
    yhat = (y - mean) * lax.rsqrt(var + 1e-5)
    y = yhat * gamma_ref[...] + beta_ref[...]
    out_ref[...] = jnp.where(y >= 0, y, 0.01 * y)


def _fire_chunk(x_hbm, idx_hbm, pt0, idx_v, rows_v, sem):
    pltpu.sync_copy(idx_hbm.at[pl.ds(pt0 * K, CHUNK * K)], idx_v)
    copies = []
    for g in range(CHUNK * K // GATHER):
        copies.append(pltpu.async_copy(
            x_hbm.at[idx_v.at[pl.ds(g * GATHER, GATHER)]],
            rows_v.at[pl.ds(g * GATHER, GATHER), :], sem))
    return copies


def _sc_gather_maxpool(x_hbm, idx_hbm, out_hbm,
                       idx_v0, idx_v1, rows_v0, rows_v1, feat_v, sem0, sem1):
    wid = lax.axis_index("s") * NC + lax.axis_index("c")
    base_pt = wid * PTS_PER_W
    idx_bufs = (idx_v0, idx_v1)
    row_bufs = (rows_v0, rows_v1)
    sems = (sem0, sem1)
    n_chunks = PTS_PER_W // CHUNK

    pending = _fire_chunk(x_hbm, idx_hbm, base_pt, idx_bufs[0], row_bufs[0],
                          sems[0])
    for t in range(n_chunks):
        pt0 = base_pt + t * CHUNK
        nxt = (t + 1) % NBUF
        if t + 1 < n_chunks:
            nxt_pending = _fire_chunk(x_hbm, idx_hbm, pt0 + CHUNK,
                                      idx_bufs[nxt], row_bufs[nxt], sems[nxt])
        for cp in pending:
            cp.wait()
        rows_v = row_bufs[t % NBUF]

        def body(p, _):
            for c4 in range(C // L):
                acc = rows_v[p * K, pl.ds(c4 * L, L)]
                for j in range(1, K):
                    acc = jnp.maximum(acc, rows_v[p * K + j, pl.ds(c4 * L, L)])
                feat_v[p, pl.ds(c4 * L, L)] = acc
            return 0

        lax.fori_loop(0, CHUNK, body, 0, unroll=False)
        pltpu.sync_copy(feat_v, out_hbm.at[pl.ds(pt0, CHUNK)])
        if t + 1 < n_chunks:
            pending = nxt_pending


_sc_gather = functools.partial(
    pl.kernel,
    mesh=plsc.VectorSubcoreMesh(core_axis_name="c", subcore_axis_name="s"),
    out_type=jax.ShapeDtypeStruct((B * N, CP), jnp.float32),
    scratch_types=[
        pltpu.VMEM((CHUNK * K,), jnp.int32),
        pltpu.VMEM((CHUNK * K,), jnp.int32),
        pltpu.VMEM((CHUNK * K, CP), jnp.float32),
        pltpu.VMEM((CHUNK * K, CP), jnp.float32),
        pltpu.VMEM((CHUNK, CP), jnp.float32),
        pltpu.SemaphoreType.DMA,
        pltpu.SemaphoreType.DMA,
    ],
)(_sc_gather_maxpool)


@jax.jit
def kernel(x, W, gamma, beta):
    x_pad = jnp.pad(x.reshape(B * N, C), ((0, 0), (0, CP - C)))
    idx = pl.pallas_call(
        _topk_idx_body,
        grid=(B, N // TILE),
        in_specs=[
            pl.BlockSpec((1, TILE, C), lambda b, i: (b, i, 0)),
            pl.BlockSpec((1, N, C), lambda b, i: (b, 0, 0)),
        ],
        out_specs=pl.BlockSpec((1, TILE, K), lambda b, i: (b, i, 0)),
        out_shape=jax.ShapeDtypeStruct((B, N, K), jnp.int32),
    )(x, x)
    feat = _sc_gather(x_pad, idx.reshape(B * N * K))
    out = pl.pallas_call(
        _head_body,
        out_shape=jax.ShapeDtypeStruct((B * N, C), jnp.float32),
    )(feat[:N], feat[N:], W, gamma.reshape(1, C), beta.reshape(1, C))
    return out.reshape(B, N, C)


# speedup vs baseline: 1.1931x; 1.1931x over previous
import functools

import jax
import jax.numpy as jnp
from jax import lax
from jax.experimental import pallas as pl
from jax.experimental.pallas import tpu as pltpu
from jax.experimental.pallas import tpu_sc as plsc

B, N, C, K = 2, 4096, 64, 16
CP = 128
TILE = 512
NEG_BIG = -1e30

_info = plsc.get_sparse_core_info()
NC, NS, L = _info.num_cores, _info.num_subcores, _info.num_lanes
NW = NC * NS
PTS_PER_W = (B * N) // NW
CHUNK = 16
GATHER = 128
NBUF = 2


def _topk_idx_body(x_rows_ref, x_all_ref, idx_ref):
        b = pl.program_id(0)
        xr = x_rows_ref[0]
        xa = x_all_ref[0]
        inner = lax.dot_general(xr, xa, (((1,), (1,)), ((), ())),
                                preferred_element_type=jnp.float32)
        xx_r = jnp.sum(xr * xr, axis=1, keepdims=True)
        xx_a = jnp.sum(xa * xa, axis=1).reshape(1, N)
        neg = 2.0 * inner - xx_r - xx_a
        colf = lax.broadcasted_iota(jnp.int32, (TILE, N), 1).astype(jnp.float32)
        BIGF = 1e9
        m = jnp.max(neg, axis=1, keepdims=True)
        picks = []
        for t in range(K):
            hit = neg == m
            amin_f = jnp.min(jnp.where(hit, colf, BIGF), axis=1,
                             keepdims=True)
            picks.append(amin_f)
            if t < K - 1:
                neg = jnp.where(hit, NEG_BIG, neg)
                m = jnp.max(neg, axis=1, keepdims=True)
        idx_f = jnp.minimum(jnp.concatenate(picks, axis=1), float(N - 1))
        idx_ref[0] = idx_f.astype(jnp.int32) + b * N


def _head_body(feat0_ref, feat1_ref, w_ref, gamma_ref, beta_ref, out_ref):
    feat = jnp.concatenate([feat0_ref[...][:, :C], feat1_ref[...][:, :C]],
                           axis=0)
    w = w_ref[...]
    y = lax.dot_general(feat, w, (((1,), (1,)), ((), ())),
                        preferred_element_type=jnp.float32)
    mean = jnp.mean(y, axis=0, keepdims=True)
    var = jnp.mean(y * y, axis=0, keepdims=True) - mean * mean
    yhat = (y - mean) * lax.rsqrt(var + 1e-5)
    y = yhat * gamma_ref[...] + beta_ref[...]
    out_ref[...] = jnp.where(y >= 0, y, 0.01 * y)


def _fire_chunk(x_hbm, idx_hbm, pt0, idx_v, rows_v, sem):
    pltpu.sync_copy(idx_hbm.at[pl.ds(pt0 * K, CHUNK * K)], idx_v)
    copies = []
    for g in range(CHUNK * K // GATHER):
        copies.append(pltpu.async_copy(
            x_hbm.at[idx_v.at[pl.ds(g * GATHER, GATHER)]],
            rows_v.at[pl.ds(g * GATHER, GATHER), :], sem))
    return copies


def _sc_gather_maxpool(x_hbm, idx_hbm, out_hbm,
                       idx_v0, idx_v1, rows_v0, rows_v1, feat_v, sem0, sem1):
    wid = lax.axis_index("s") * NC + lax.axis_index("c")
    base_pt = wid * PTS_PER_W
    idx_bufs = (idx_v0, idx_v1)
    row_bufs = (rows_v0, rows_v1)
    sems = (sem0, sem1)
    n_chunks = PTS_PER_W // CHUNK

    pending = _fire_chunk(x_hbm, idx_hbm, base_pt, idx_bufs[0], row_bufs[0],
                          sems[0])
    for t in range(n_chunks):
        pt0 = base_pt + t * CHUNK
        nxt = (t + 1) % NBUF
        if t + 1 < n_chunks:
            nxt_pending = _fire_chunk(x_hbm, idx_hbm, pt0 + CHUNK,
                                      idx_bufs[nxt], row_bufs[nxt], sems[nxt])
        for cp in pending:
            cp.wait()
        rows_v = row_bufs[t % NBUF]

        def body(p, _):
            for c4 in range(C // L):
                acc = rows_v[p * K, pl.ds(c4 * L, L)]
                for j in range(1, K):
                    acc = jnp.maximum(acc, rows_v[p * K + j, pl.ds(c4 * L, L)])
                feat_v[p, pl.ds(c4 * L, L)] = acc
            return 0

        lax.fori_loop(0, CHUNK, body, 0, unroll=4)
        pltpu.sync_copy(feat_v, out_hbm.at[pl.ds(pt0, CHUNK)])
        if t + 1 < n_chunks:
            pending = nxt_pending


_sc_gather = functools.partial(
    pl.kernel,
    mesh=plsc.VectorSubcoreMesh(core_axis_name="c", subcore_axis_name="s"),
    out_type=jax.ShapeDtypeStruct((B * N, CP), jnp.float32),
    scratch_types=[
        pltpu.VMEM((CHUNK * K,), jnp.int32),
        pltpu.VMEM((CHUNK * K,), jnp.int32),
        pltpu.VMEM((CHUNK * K, CP), jnp.float32),
        pltpu.VMEM((CHUNK * K, CP), jnp.float32),
        pltpu.VMEM((CHUNK, CP), jnp.float32),
        pltpu.SemaphoreType.DMA,
        pltpu.SemaphoreType.DMA,
    ],
)(_sc_gather_maxpool)


@jax.jit
def kernel(x, W, gamma, beta):
    x_pad = jnp.pad(x.reshape(B * N, C), ((0, 0), (0, CP - C)))
    idx = pl.pallas_call(
        _topk_idx_body,
        grid=(B, N // TILE),
        in_specs=[
            pl.BlockSpec((1, TILE, C), lambda b, i: (b, i, 0)),
            pl.BlockSpec((1, N, C), lambda b, i: (b, 0, 0)),
        ],
        out_specs=pl.BlockSpec((1, TILE, K), lambda b, i: (b, i, 0)),
        out_shape=jax.ShapeDtypeStruct((B, N, K), jnp.int32),
    )(x, x)
    feat = _sc_gather(x_pad, idx.reshape(B * N * K))
    out = pl.pallas_call(
        _head_body,
        out_shape=jax.ShapeDtypeStruct((B * N, C), jnp.float32),
    )(feat[:N], feat[N:], W, gamma.reshape(1, C), beta.reshape(1, C))
    return out.reshape(B, N, C)


# speedup vs baseline: 1.2070x; 1.0116x over previous
import functools

import jax
import jax.numpy as jnp
from jax import lax
from jax.experimental import pallas as pl
from jax.experimental.pallas import tpu as pltpu
from jax.experimental.pallas import tpu_sc as plsc

B, N, C, K = 2, 4096, 64, 16
CP = 128
TILE = 512
NEG_BIG = -1e30

_info = plsc.get_sparse_core_info()
NC, NS, L = _info.num_cores, _info.num_subcores, _info.num_lanes
NW = NC * NS
PTS_PER_W = (B * N) // NW
CHUNK = 16
GATHER = 128
NBUF = 2


def _topk_idx_body(x_rows_ref, x_all_ref, idx_ref, xpad_ref):
        b = pl.program_id(0)
        xr = x_rows_ref[0]
        xa = x_all_ref[0]
        xpad_ref[0] = jnp.concatenate(
            [xr, jnp.zeros((TILE, CP - C), jnp.float32)], axis=1)
        inner = lax.dot_general(xr, xa, (((1,), (1,)), ((), ())),
                                preferred_element_type=jnp.float32)
        xx_r = jnp.sum(xr * xr, axis=1, keepdims=True)
        xx_a = jnp.sum(xa * xa, axis=1).reshape(1, N)
        neg = 2.0 * inner - xx_r - xx_a
        colf = lax.broadcasted_iota(jnp.int32, (TILE, N), 1).astype(jnp.float32)
        BIGF = 1e9
        m = jnp.max(neg, axis=1, keepdims=True)
        picks = []
        for t in range(K):
            hit = neg == m
            amin_f = jnp.min(jnp.where(hit, colf, BIGF), axis=1,
                             keepdims=True)
            picks.append(amin_f)
            if t < K - 1:
                neg = jnp.where(hit, NEG_BIG, neg)
                m = jnp.max(neg, axis=1, keepdims=True)
        idx_f = jnp.minimum(jnp.concatenate(picks, axis=1), float(N - 1))
        idx_ref[0] = idx_f.astype(jnp.int32) + b * N


def _head_body(feat0_ref, feat1_ref, w_ref, gamma_ref, beta_ref, out_ref):
    feat = jnp.concatenate([feat0_ref[...][:, :C], feat1_ref[...][:, :C]],
                           axis=0)
    w = w_ref[...]
    y = lax.dot_general(feat, w, (((1,), (1,)), ((), ())),
                        preferred_element_type=jnp.float32)
    mean = jnp.mean(y, axis=0, keepdims=True)
    var = jnp.mean(y * y, axis=0, keepdims=True) - mean * mean
    yhat = (y - mean) * lax.rsqrt(var + 1e-5)
    y = yhat * gamma_ref[...] + beta_ref[...]
    out_ref[...] = jnp.where(y >= 0, y, 0.01 * y)


def _fire_chunk(x_hbm, idx_hbm, pt0, idx_v, rows_v, sem):
    pltpu.sync_copy(idx_hbm.at[pl.ds(pt0 * K, CHUNK * K)], idx_v)
    copies = []
    for g in range(CHUNK * K // GATHER):
        copies.append(pltpu.async_copy(
            x_hbm.at[idx_v.at[pl.ds(g * GATHER, GATHER)]],
            rows_v.at[pl.ds(g * GATHER, GATHER), :], sem))
    return copies


def _sc_gather_maxpool(x_hbm, idx_hbm, out_hbm,
                       idx_v0, idx_v1, rows_v0, rows_v1, feat_v, sem0, sem1):
    wid = lax.axis_index("s") * NC + lax.axis_index("c")
    base_pt = wid * PTS_PER_W
    idx_bufs = (idx_v0, idx_v1)
    row_bufs = (rows_v0, rows_v1)
    sems = (sem0, sem1)
    n_chunks = PTS_PER_W // CHUNK

    pending = _fire_chunk(x_hbm, idx_hbm, base_pt, idx_bufs[0], row_bufs[0],
                          sems[0])
    for t in range(n_chunks):
        pt0 = base_pt + t * CHUNK
        nxt = (t + 1) % NBUF
        if t + 1 < n_chunks:
            nxt_pending = _fire_chunk(x_hbm, idx_hbm, pt0 + CHUNK,
                                      idx_bufs[nxt], row_bufs[nxt], sems[nxt])
        for cp in pending:
            cp.wait()
        rows_v = row_bufs[t % NBUF]

        def body(p, _):
            for c4 in range(C // L):
                acc = rows_v[p * K, pl.ds(c4 * L, L)]
                for j in range(1, K):
                    acc = jnp.maximum(acc, rows_v[p * K + j, pl.ds(c4 * L, L)])
                feat_v[p, pl.ds(c4 * L, L)] = acc
            return 0

        lax.fori_loop(0, CHUNK, body, 0, unroll=False)
        pltpu.sync_copy(feat_v, out_hbm.at[pl.ds(pt0, CHUNK)])
        if t + 1 < n_chunks:
            pending = nxt_pending


_sc_gather = functools.partial(
    pl.kernel,
    mesh=plsc.VectorSubcoreMesh(core_axis_name="c", subcore_axis_name="s"),
    out_type=jax.ShapeDtypeStruct((B * N, CP), jnp.float32),
    scratch_types=[
        pltpu.VMEM((CHUNK * K,), jnp.int32),
        pltpu.VMEM((CHUNK * K,), jnp.int32),
        pltpu.VMEM((CHUNK * K, CP), jnp.float32),
        pltpu.VMEM((CHUNK * K, CP), jnp.float32),
        pltpu.VMEM((CHUNK, CP), jnp.float32),
        pltpu.SemaphoreType.DMA,
        pltpu.SemaphoreType.DMA,
    ],
)(_sc_gather_maxpool)


@jax.jit
def kernel(x, W, gamma, beta):
    idx, x_pad = pl.pallas_call(
        _topk_idx_body,
        grid=(B, N // TILE),
        in_specs=[
            pl.BlockSpec((1, TILE, C), lambda b, i: (b, i, 0)),
            pl.BlockSpec((1, N, C), lambda b, i: (b, 0, 0)),
        ],
        out_specs=[
            pl.BlockSpec((1, TILE, K), lambda b, i: (b, i, 0)),
            pl.BlockSpec((1, TILE, CP), lambda b, i: (b, i, 0)),
        ],
        out_shape=[
            jax.ShapeDtypeStruct((B, N, K), jnp.int32),
            jax.ShapeDtypeStruct((B, N, CP), jnp.float32),
        ],
    )(x, x)
    feat = _sc_gather(x_pad.reshape(B * N, CP), idx.reshape(B * N * K))
    out = pl.pallas_call(
        _head_body,
        out_shape=jax.ShapeDtypeStruct((B * N, C), jnp.float32),
    )(feat[:N], feat[N:], W, gamma.reshape(1, C), beta.reshape(1, C))
    return out.reshape(B, N, C)
